# baseline (speedup 1.0000x reference)
import functools

import jax
import jax.numpy as jnp
from jax.experimental import pallas as pl
from jax.experimental.pallas import tpu as pltpu

_B = 128
_D = 5
_N = 31
_OP = 16
_PRED = 512
_FEAT = 64
_HID = 128
_BITMAP = 1000
_REP = 128

_NDATA = 6


def _dot(a, b):
    return jax.lax.dot_general(
        a, b, (((1,), (0,)), ((), ())), preferred_element_type=jnp.float32
    )


def _tree_body(
    op_hbm, feat_hbm, c1_hbm, c2_hbm, bm_hbm, hc_hbm,
    WopE_ref, WfeatE_ref, Wp_ref, A2_ref, A3_ref, Wbm_ref, bbm_ref, A4_ref,
    biasE_ref, Wch_ref, W2_ref, b2_ref, W3_ref, b3_ref,
    W_h21_ref, b_h21_ref, W_h31_ref, b_h31_ref, W_o1_ref, b_o1_ref,
    W_h22_ref, b_h22_ref, W_h32_ref, b_h32_ref, W_o2_ref, b_o2_ref,
    cost_ref, card_ref,
    op_buf, feat_buf, c1_buf, c2_buf, bm_buf, hc_buf, sems, z_sc,
):
    i = pl.program_id(0)
    slot = jax.lax.rem(i, 2)
    nslot = jax.lax.rem(i + 1, 2)
    hbms = [op_hbm, feat_hbm, c1_hbm, c2_hbm, bm_hbm, hc_hbm]
    bufs = [op_buf, feat_buf, c1_buf, c2_buf, bm_buf, hc_buf]

    def start_fetch(node, s):
        for k in range(_NDATA):
            pltpu.make_async_copy(
                hbms[k].at[:, node, :], bufs[k].at[s], sems.at[s, k]
            ).start()

    @pl.when(i == 0)
    def _warmup():
        start_fetch(0, 0)

    @pl.when(i + 1 < _N)
    def _prefetch():
        start_fetch(i + 1, nslot)

    for k in range(_NDATA):
        pltpu.make_async_copy(
            hbms[k].at[:, i, :], bufs[k].at[slot], sems.at[slot, k]
        ).wait()

    c1 = _dot(c1_buf[slot], Wp_ref[...])
    c2 = _dot(c2_buf[slot], Wp_ref[...])
    bmE = (_dot(bm_buf[slot], Wbm_ref[...]) + bbm_ref[...]) * hc_buf[slot]
    z = _dot(op_buf[slot], WopE_ref[...])
    z = z + _dot(feat_buf[slot], WfeatE_ref[...])
    z = z + _dot(c1, A2_ref[...])
    z = z + _dot(c2, A3_ref[...])
    z = z + _dot(bmE, A4_ref[...])
    z = z + biasE_ref[...]
    z_sc[pl.ds(i * _B, _B), :] = z

    @pl.when(i == _N - 1)
    def _stage2():
        Wch = Wch_ref[...]
        Wlr = Wch[0 * _REP:1 * _REP]
        Wrr = Wch[1 * _REP:2 * _REP]
        Wll = Wch[2 * _REP:3 * _REP]
        Wlrt = Wch[3 * _REP:4 * _REP]
        Wrl = Wch[4 * _REP:5 * _REP]
        Wrrt = Wch[5 * _REP:6 * _REP]
        W2 = W2_ref[...]
        b2 = b2_ref[...]
        W3 = W3_ref[...]
        b3 = b3_ref[...]

        reps = [None] * _D
        for l in range(_D - 1, -1, -1):
            n = 1 << l
            a = n - 1
            zl = z_sc[a * _B:(a + n) * _B, :]
            if l <= _D - 2:
                C = reps[l + 1].reshape(n, 2, _B, _REP)
                left = C[:, 0].reshape(n * _B, _REP)
                right = C[:, 1].reshape(n * _B, _REP)
                zl = zl + _dot(left, Wlr) + _dot(right, Wrr)
            if l <= _D - 3:
                G = reps[l + 2].reshape(n, 4, _B, _REP)
                zl = (zl
                      + _dot(G[:, 0].reshape(n * _B, _REP), Wll)
                      + _dot(G[:, 1].reshape(n * _B, _REP), Wlrt)
                      + _dot(G[:, 2].reshape(n * _B, _REP), Wrl)
                      + _dot(G[:, 3].reshape(n * _B, _REP), Wrrt))
            h = jnp.maximum(zl, 0.0)
            h = jnp.maximum(_dot(h, W2) + b2, 0.0)
            h = jnp.maximum(_dot(h, W3) + b3, 0.0)
            reps[l] = h

        root = reps[0]
        cost = jnp.maximum(_dot(root, W_h21_ref[...]) + b_h21_ref[...], 0.0)
        cost = jnp.maximum(_dot(cost, W_h31_ref[...]) + b_h31_ref[...], 0.0)
        cost_ref[...] = jax.nn.sigmoid(_dot(cost, W_o1_ref[...]) + b_o1_ref[...])
        card = jnp.maximum(_dot(root, W_h22_ref[...]) + b_h22_ref[...], 0.0)
        card = jnp.maximum(_dot(card, W_h32_ref[...]) + b_h32_ref[...], 0.0)
        card_ref[...] = jax.nn.sigmoid(_dot(card, W_o2_ref[...]) + b_o2_ref[...])


@jax.jit
def kernel(op_x, feat_x, cond1_x, cond2_x, bitmap_x, has_cond,
           W_op, b_op, W_pred, b_pred, W_bm, b_bm, W_feat, b_feat,
           W_r1, b_r1, W_r2, b_r2, W_r3, b_r3,
           W_h21, b_h21, W_h31, b_h31, W_o1, b_o1,
           W_h22, b_h22, W_h32, b_h32, W_o2, b_o2):
    A0 = W_r1[0 * _HID:1 * _HID]
    A1 = W_r1[1 * _HID:2 * _HID]
    A2 = W_r1[2 * _HID:3 * _HID]
    A3 = W_r1[3 * _HID:4 * _HID]
    A4 = W_r1[4 * _HID:5 * _HID]
    Wch = W_r1[5 * _HID:]
    WopE = W_op @ A0
    WfeatE = W_feat @ A1
    biasE = (b_r1 + b_op @ A0 + b_feat @ A1 + b_pred @ A2 + b_pred @ A3)

    hcb = jnp.broadcast_to(has_cond[:, :, None], (_B, _N, _HID))

    b2d = lambda b: b.reshape(1, -1)

    data = [op_x, feat_x, cond1_x, cond2_x, bitmap_x, hcb]
    weights = [WopE, WfeatE, W_pred, A2, A3, W_bm, b2d(b_bm), A4,
               b2d(biasE), Wch, W_r2, b2d(b_r2), W_r3, b2d(b_r3),
               W_h21, b2d(b_h21), W_h31, b2d(b_h31), W_o1, b2d(b_o1),
               W_h22, b2d(b_h22), W_h32, b2d(b_h32), W_o2, b2d(b_o2)]

    hbm_spec = pl.BlockSpec(memory_space=pltpu.MemorySpace.HBM)

    def w_spec(shape):
        nd = len(shape)
        return pl.BlockSpec(tuple(shape), lambda i, _nd=nd: (0,) * _nd)

    in_specs = [hbm_spec] * _NDATA + [w_spec(w.shape) for w in weights]

    out_shape = (
        jax.ShapeDtypeStruct((_B, 1), jnp.float32),
        jax.ShapeDtypeStruct((_B, 1), jnp.float32),
    )
    out_specs = (
        pl.BlockSpec((_B, 1), lambda i: (0, 0)),
        pl.BlockSpec((_B, 1), lambda i: (0, 0)),
    )

    scratch_shapes = [
        pltpu.VMEM((2, _B, _OP), jnp.float32),
        pltpu.VMEM((2, _B, _FEAT), jnp.float32),
        pltpu.VMEM((2, _B, _PRED), jnp.float32),
        pltpu.VMEM((2, _B, _PRED), jnp.float32),
        pltpu.VMEM((2, _B, _BITMAP), jnp.float32),
        pltpu.VMEM((2, _B, _HID), jnp.float32),
        pltpu.SemaphoreType.DMA((2, _NDATA)),
        pltpu.VMEM((_N * _B, 512), jnp.float32),
    ]

    cost, card = pl.pallas_call(
        _tree_body,
        grid=(_N,),
        in_specs=in_specs,
        out_specs=out_specs,
        out_shape=out_shape,
        scratch_shapes=scratch_shapes,
        compiler_params=pltpu.CompilerParams(
            dimension_semantics=("arbitrary",),
        ),
    )(*data, *weights)
    return (cost, card)


# speedup vs baseline: 1.0018x; 1.0018x over previous
import functools

import jax
import jax.numpy as jnp
from jax.experimental import pallas as pl
from jax.experimental.pallas import tpu as pltpu

_B = 128
_D = 5
_N = 31
_OP = 16
_PRED = 512
_FEAT = 64
_HID = 128
_BITMAP = 1000
_REP = 128

_NDATA = 6


def _dot(a, b):
    return jax.lax.dot_general(
        a.astype(jnp.bfloat16), b.astype(jnp.bfloat16),
        (((1,), (0,)), ((), ())), preferred_element_type=jnp.float32
    )


def _dot32(a, b):
    return jax.lax.dot_general(
        a, b, (((1,), (0,)), ((), ())), preferred_element_type=jnp.float32
    )


def _tree_body(
    op_hbm, feat_hbm, c1_hbm, c2_hbm, bm_hbm, hc_hbm,
    WopE_ref, WfeatE_ref, Wp_ref, A2_ref, A3_ref, Wbm_ref, bbm_ref, A4_ref,
    biasE_ref, Wch_ref, W2_ref, b2_ref, W3_ref, b3_ref,
    W_h21_ref, b_h21_ref, W_h31_ref, b_h31_ref, W_o1_ref, b_o1_ref,
    W_h22_ref, b_h22_ref, W_h32_ref, b_h32_ref, W_o2_ref, b_o2_ref,
    cost_ref, card_ref,
    op_buf, feat_buf, c1_buf, c2_buf, bm_buf, hc_buf, sems, z_sc,
):
    i = pl.program_id(0)
    slot = jax.lax.rem(i, 2)
    nslot = jax.lax.rem(i + 1, 2)
    hbms = [op_hbm, feat_hbm, c1_hbm, c2_hbm, bm_hbm, hc_hbm]
    bufs = [op_buf, feat_buf, c1_buf, c2_buf, bm_buf, hc_buf]

    def start_fetch(node, s):
        for k in range(_NDATA):
            pltpu.make_async_copy(
                hbms[k].at[:, node, :], bufs[k].at[s], sems.at[s, k]
            ).start()

    @pl.when(i == 0)
    def _warmup():
        start_fetch(0, 0)

    @pl.when(i + 1 < _N)
    def _prefetch():
        start_fetch(i + 1, nslot)

    for k in range(_NDATA):
        pltpu.make_async_copy(
            hbms[k].at[:, i, :], bufs[k].at[slot], sems.at[slot, k]
        ).wait()

    c1 = _dot(c1_buf[slot], Wp_ref[...])
    c2 = _dot(c2_buf[slot], Wp_ref[...])
    bmE = (_dot(bm_buf[slot], Wbm_ref[...]) + bbm_ref[...]) * hc_buf[slot]
    z = _dot(op_buf[slot], WopE_ref[...])
    z = z + _dot(feat_buf[slot], WfeatE_ref[...])
    z = z + _dot(c1, A2_ref[...])
    z = z + _dot(c2, A3_ref[...])
    z = z + _dot(bmE, A4_ref[...])
    z = z + biasE_ref[...]
    z_sc[pl.ds(i * _B, _B), :] = z

    @pl.when(i == _N - 1)
    def _stage2():
        Wch = Wch_ref[...]
        Wlr = Wch[0 * _REP:1 * _REP]
        Wrr = Wch[1 * _REP:2 * _REP]
        Wll = Wch[2 * _REP:3 * _REP]
        Wlrt = Wch[3 * _REP:4 * _REP]
        Wrl = Wch[4 * _REP:5 * _REP]
        Wrrt = Wch[5 * _REP:6 * _REP]
        W2 = W2_ref[...]
        b2 = b2_ref[...]
        W3 = W3_ref[...]
        b3 = b3_ref[...]

        reps = [None] * _D
        for l in range(_D - 1, -1, -1):
            dot = _dot if l >= _D - 2 else _dot32
            n = 1 << l
            a = n - 1
            zl = z_sc[a * _B:(a + n) * _B, :]
            if l <= _D - 2:
                C = reps[l + 1].reshape(n, 2, _B, _REP)
                left = C[:, 0].reshape(n * _B, _REP)
                right = C[:, 1].reshape(n * _B, _REP)
                zl = zl + dot(left, Wlr) + dot(right, Wrr)
            if l <= _D - 3:
                G = reps[l + 2].reshape(n, 4, _B, _REP)
                zl = (zl
                      + dot(G[:, 0].reshape(n * _B, _REP), Wll)
                      + dot(G[:, 1].reshape(n * _B, _REP), Wlrt)
                      + dot(G[:, 2].reshape(n * _B, _REP), Wrl)
                      + dot(G[:, 3].reshape(n * _B, _REP), Wrrt))
            h = jnp.maximum(zl, 0.0)
            h = jnp.maximum(dot(h, W2) + b2, 0.0)
            h = jnp.maximum(dot(h, W3) + b3, 0.0)
            reps[l] = h

        root = reps[0]
        cost = jnp.maximum(_dot32(root, W_h21_ref[...]) + b_h21_ref[...], 0.0)
        cost = jnp.maximum(_dot32(cost, W_h31_ref[...]) + b_h31_ref[...], 0.0)
        cost_ref[...] = jax.nn.sigmoid(_dot32(cost, W_o1_ref[...]) + b_o1_ref[...])
        card = jnp.maximum(_dot32(root, W_h22_ref[...]) + b_h22_ref[...], 0.0)
        card = jnp.maximum(_dot32(card, W_h32_ref[...]) + b_h32_ref[...], 0.0)
        card_ref[...] = jax.nn.sigmoid(_dot32(card, W_o2_ref[...]) + b_o2_ref[...])


@jax.jit
def kernel(op_x, feat_x, cond1_x, cond2_x, bitmap_x, has_cond,
           W_op, b_op, W_pred, b_pred, W_bm, b_bm, W_feat, b_feat,
           W_r1, b_r1, W_r2, b_r2, W_r3, b_r3,
           W_h21, b_h21, W_h31, b_h31, W_o1, b_o1,
           W_h22, b_h22, W_h32, b_h32, W_o2, b_o2):
    A0 = W_r1[0 * _HID:1 * _HID]
    A1 = W_r1[1 * _HID:2 * _HID]
    A2 = W_r1[2 * _HID:3 * _HID]
    A3 = W_r1[3 * _HID:4 * _HID]
    A4 = W_r1[4 * _HID:5 * _HID]
    Wch = W_r1[5 * _HID:]
    WopE = W_op @ A0
    WfeatE = W_feat @ A1
    biasE = (b_r1 + b_op @ A0 + b_feat @ A1 + b_pred @ A2 + b_pred @ A3)

    hcb = jnp.broadcast_to(has_cond[:, :, None], (_B, _N, _HID))

    b2d = lambda b: b.reshape(1, -1)

    data = [op_x, feat_x, cond1_x, cond2_x, bitmap_x, hcb]
    weights = [WopE, WfeatE, W_pred, A2, A3, W_bm, b2d(b_bm), A4,
               b2d(biasE), Wch, W_r2, b2d(b_r2), W_r3, b2d(b_r3),
               W_h21, b2d(b_h21), W_h31, b2d(b_h31), W_o1, b2d(b_o1),
               W_h22, b2d(b_h22), W_h32, b2d(b_h32), W_o2, b2d(b_o2)]

    hbm_spec = pl.BlockSpec(memory_space=pltpu.MemorySpace.HBM)

    def w_spec(shape):
        nd = len(shape)
        return pl.BlockSpec(tuple(shape), lambda i, _nd=nd: (0,) * _nd)

    in_specs = [hbm_spec] * _NDATA + [w_spec(w.shape) for w in weights]

    out_shape = (
        jax.ShapeDtypeStruct((_B, 1), jnp.float32),
        jax.ShapeDtypeStruct((_B, 1), jnp.float32),
    )
    out_specs = (
        pl.BlockSpec((_B, 1), lambda i: (0, 0)),
        pl.BlockSpec((_B, 1), lambda i: (0, 0)),
    )

    scratch_shapes = [
        pltpu.VMEM((2, _B, _OP), jnp.float32),
        pltpu.VMEM((2, _B, _FEAT), jnp.float32),
        pltpu.VMEM((2, _B, _PRED), jnp.float32),
        pltpu.VMEM((2, _B, _PRED), jnp.float32),
        pltpu.VMEM((2, _B, _BITMAP), jnp.float32),
        pltpu.VMEM((2, _B, _HID), jnp.float32),
        pltpu.SemaphoreType.DMA((2, _NDATA)),
        pltpu.VMEM((_N * _B, 512), jnp.float32),
    ]

    cost, card = pl.pallas_call(
        _tree_body,
        grid=(_N,),
        in_specs=in_specs,
        out_specs=out_specs,
        out_shape=out_shape,
        scratch_shapes=scratch_shapes,
        compiler_params=pltpu.CompilerParams(
            dimension_semantics=("arbitrary",),
        ),
    )(*data, *weights)
    return (cost, card)


# speedup vs baseline: 1.1032x; 1.1012x over previous
import functools

import jax
import jax.numpy as jnp
from jax.experimental import pallas as pl
from jax.experimental.pallas import tpu as pltpu

_B = 128
_D = 5
_N = 31
_OP = 16
_PRED = 512
_FEAT = 64
_HID = 128
_BITMAP = 1000
_REP = 128

_NDATA = 6


def _dot(a, b):
    return jax.lax.dot_general(
        a.astype(jnp.bfloat16), b.astype(jnp.bfloat16),
        (((1,), (0,)), ((), ())), preferred_element_type=jnp.float32
    )


def _dot32(a, b):
    return jax.lax.dot_general(
        a, b, (((1,), (0,)), ((), ())), preferred_element_type=jnp.float32
    )


def _row(b_ref):
    return b_ref[...].reshape(1, -1)


def _tree_body(
    op_hbm, feat_hbm, c1_hbm, c2_hbm, bm_hbm, hc_hbm,
    W_op_ref, b_op_ref, Wp_ref, bp_ref, Wbm_ref, bbm_ref,
    W_feat_ref, b_feat_ref, Wr1_ref, br1_ref, W2_ref, b2_ref,
    W3_ref, b3_ref,
    W_h21_ref, b_h21_ref, W_h31_ref, b_h31_ref, W_o1_ref, b_o1_ref,
    W_h22_ref, b_h22_ref, W_h32_ref, b_h32_ref, W_o2_ref, b_o2_ref,
    cost_ref, card_ref,
    op_buf, feat_buf, c1_buf, c2_buf, bm_buf, hc_buf, sems, z_sc,
):
    i = pl.program_id(0)
    slot = jax.lax.rem(i, 2)
    nslot = jax.lax.rem(i + 1, 2)
    hbms = [op_hbm, feat_hbm, c1_hbm, c2_hbm, bm_hbm]
    bufs = [op_buf, feat_buf, c1_buf, c2_buf, bm_buf]

    def start_fetch(node, s):
        for k in range(_NDATA - 1):
            pltpu.make_async_copy(
                hbms[k].at[:, node, :], bufs[k].at[s], sems.at[s, k]
            ).start()
        pltpu.make_async_copy(
            hc_hbm.at[:, node, :], hc_buf.at[s], sems.at[s, _NDATA - 1]
        ).start()

    @pl.when(i == 0)
    def _warmup():
        start_fetch(0, 0)

    @pl.when(i + 1 < _N)
    def _prefetch():
        start_fetch(i + 1, nslot)

    for k in range(_NDATA - 1):
        pltpu.make_async_copy(
            hbms[k].at[:, i, :], bufs[k].at[slot], sems.at[slot, k]
        ).wait()
    pltpu.make_async_copy(
        hc_hbm.at[:, i, :], hc_buf.at[slot], sems.at[slot, _NDATA - 1]
    ).wait()

    Wr1 = Wr1_ref[...]
    A0 = Wr1[0 * _HID:1 * _HID]
    A1 = Wr1[1 * _HID:2 * _HID]
    A2 = Wr1[2 * _HID:3 * _HID]
    A3 = Wr1[3 * _HID:4 * _HID]
    A4 = Wr1[4 * _HID:5 * _HID]

    op_v = _dot(op_buf[slot], W_op_ref[...]) + _row(b_op_ref)
    feat_v = _dot(feat_buf[slot], W_feat_ref[...]) + _row(b_feat_ref)
    bp = _row(bp_ref)
    c1 = _dot(c1_buf[slot], Wp_ref[...]) + bp
    c2 = _dot(c2_buf[slot], Wp_ref[...]) + bp
    bmE = (_dot(bm_buf[slot], Wbm_ref[...]) + _row(bbm_ref)) * hc_buf[slot]
    z = _dot(op_v, A0)
    z = z + _dot(feat_v, A1)
    z = z + _dot(c1, A2)
    z = z + _dot(c2, A3)
    z = z + _dot(bmE, A4)
    z = z + _row(br1_ref)
    z_sc[pl.ds(i * _B, _B), :] = z

    @pl.when(i == _N - 1)
    def _stage2():
        Wch = Wr1_ref[...]
        cb = 5 * _HID
        Wlr = Wch[cb + 0 * _REP:cb + 1 * _REP]
        Wrr = Wch[cb + 1 * _REP:cb + 2 * _REP]
        Wll = Wch[cb + 2 * _REP:cb + 3 * _REP]
        Wlrt = Wch[cb + 3 * _REP:cb + 4 * _REP]
        Wrl = Wch[cb + 4 * _REP:cb + 5 * _REP]
        Wrrt = Wch[cb + 5 * _REP:cb + 6 * _REP]
        W2 = W2_ref[...]
        b2 = _row(b2_ref)
        W3 = W3_ref[...]
        b3 = _row(b3_ref)

        reps = [None] * _D
        for l in range(_D - 1, -1, -1):
            dot = _dot if l >= _D - 2 else _dot32
            n = 1 << l
            a = n - 1
            zl = z_sc[a * _B:(a + n) * _B, :]
            if l <= _D - 2:
                C = reps[l + 1].reshape(n, 2, _B, _REP)
                left = C[:, 0].reshape(n * _B, _REP)
                right = C[:, 1].reshape(n * _B, _REP)
                zl = zl + dot(left, Wlr) + dot(right, Wrr)
            if l <= _D - 3:
                G = reps[l + 2].reshape(n, 4, _B, _REP)
                zl = (zl
                      + dot(G[:, 0].reshape(n * _B, _REP), Wll)
                      + dot(G[:, 1].reshape(n * _B, _REP), Wlrt)
                      + dot(G[:, 2].reshape(n * _B, _REP), Wrl)
                      + dot(G[:, 3].reshape(n * _B, _REP), Wrrt))
            h = jnp.maximum(zl, 0.0)
            h = jnp.maximum(dot(h, W2) + b2, 0.0)
            h = jnp.maximum(dot(h, W3) + b3, 0.0)
            reps[l] = h

        root = reps[0]
        cost = jnp.maximum(_dot32(root, W_h21_ref[...]) + _row(b_h21_ref), 0.0)
        cost = jnp.maximum(_dot32(cost, W_h31_ref[...]) + _row(b_h31_ref), 0.0)
        cost_ref[...] = jax.nn.sigmoid(_dot32(cost, W_o1_ref[...]) + _row(b_o1_ref))
        card = jnp.maximum(_dot32(root, W_h22_ref[...]) + _row(b_h22_ref), 0.0)
        card = jnp.maximum(_dot32(card, W_h32_ref[...]) + _row(b_h32_ref), 0.0)
        card_ref[...] = jax.nn.sigmoid(_dot32(card, W_o2_ref[...]) + _row(b_o2_ref))


@jax.jit
def kernel(op_x, feat_x, cond1_x, cond2_x, bitmap_x, has_cond,
           W_op, b_op, W_pred, b_pred, W_bm, b_bm, W_feat, b_feat,
           W_r1, b_r1, W_r2, b_r2, W_r3, b_r3,
           W_h21, b_h21, W_h31, b_h31, W_o1, b_o1,
           W_h22, b_h22, W_h32, b_h32, W_o2, b_o2):
    hcb = jnp.broadcast_to(has_cond[:, :, None], (_B, _N, _HID))
    data = [op_x, feat_x, cond1_x, cond2_x, bitmap_x, hcb]
    weights = [W_op, b_op, W_pred, b_pred, W_bm, b_bm, W_feat, b_feat,
               W_r1, b_r1, W_r2, b_r2, W_r3, b_r3,
               W_h21, b_h21, W_h31, b_h31, W_o1, b_o1,
               W_h22, b_h22, W_h32, b_h32, W_o2, b_o2]

    hbm_spec = pl.BlockSpec(memory_space=pltpu.MemorySpace.HBM)

    def w_spec(shape):
        nd = len(shape)
        return pl.BlockSpec(tuple(shape), lambda i, _nd=nd: (0,) * _nd)

    in_specs = [hbm_spec] * _NDATA + [w_spec(w.shape) for w in weights]

    out_shape = (
        jax.ShapeDtypeStruct((_B, 1), jnp.float32),
        jax.ShapeDtypeStruct((_B, 1), jnp.float32),
    )
    out_specs = (
        pl.BlockSpec((_B, 1), lambda i: (0, 0)),
        pl.BlockSpec((_B, 1), lambda i: (0, 0)),
    )

    scratch_shapes = [
        pltpu.VMEM((2, _B, _OP), jnp.float32),
        pltpu.VMEM((2, _B, _FEAT), jnp.float32),
        pltpu.VMEM((2, _B, _PRED), jnp.float32),
        pltpu.VMEM((2, _B, _PRED), jnp.float32),
        pltpu.VMEM((2, _B, _BITMAP), jnp.float32),
        pltpu.VMEM((2, _B, _HID), jnp.float32),
        pltpu.SemaphoreType.DMA((2, _NDATA)),
        pltpu.VMEM((_N * _B, 512), jnp.float32),
    ]

    cost, card = pl.pallas_call(
        _tree_body,
        grid=(_N,),
        in_specs=in_specs,
        out_specs=out_specs,
        out_shape=out_shape,
        scratch_shapes=scratch_shapes,
        compiler_params=pltpu.CompilerParams(
            dimension_semantics=("arbitrary",),
        ),
    )(*data, *weights)
    return (cost, card)
